# baseline (speedup 1.0000x reference)
import functools
import jax
import jax.numpy as jnp
from jax import lax
from jax.experimental import pallas as pl
from jax.experimental.pallas import tpu as pltpu, tpu_sc as plsc

B, T, D = 4, 8192, 1024
NC, NS = 2, 16
NW = NC * NS
TW = T // NW
BTR = 8
NB = TW // BTR
BLK = BTR * D
L = 16

_mesh = plsc.VectorSubcoreMesh(core_axis_name="c", subcore_axis_name="s")


def _sc_add(x_hbm, pe_hbm, out_hbm, pe_v, x_v):
    wid = lax.axis_index("s") * NC + lax.axis_index("c")
    base = wid * (TW * D)

    def do_block(blk, _):
        off = base + blk * BLK
        pltpu.sync_copy(pe_hbm.at[pl.ds(off, BLK)], pe_v)
        pltpu.sync_copy(x_hbm.at[:, pl.ds(off, BLK)], x_v)

        def chunk(j, _):
            o = j * L
            pe_vec = pe_v[pl.ds(o, L)]
            for b in range(B):
                x_v[b, pl.ds(o, L)] += pe_vec
            return 0

        lax.fori_loop(0, BLK // L, chunk, 0, unroll=4)
        pltpu.sync_copy(x_v, out_hbm.at[:, pl.ds(off, BLK)])
        return 0

    lax.fori_loop(0, NB, do_block, 0)


_sc_call = functools.partial(
    pl.kernel,
    out_type=jax.ShapeDtypeStruct((B, T * D), jnp.float32),
    mesh=_mesh,
    scratch_types=[
        pltpu.VMEM((BLK,), jnp.float32),
        pltpu.VMEM((B, BLK), jnp.float32),
    ],
)(_sc_add)


def kernel(x, pe_table):
    out = _sc_call(x.reshape(B, T * D), pe_table[:T].reshape(T * D))
    return out.reshape(B, T, D)


# speedup vs baseline: 2.3434x; 2.3434x over previous
import functools
import jax
import jax.numpy as jnp
from jax import lax
from jax.experimental import pallas as pl
from jax.experimental.pallas import tpu as pltpu, tpu_sc as plsc

B, T, D = 4, 8192, 1024
NC, NS = 2, 16
NW = NC * NS
TW = T // NW
BTR = 8
NB = TW // BTR
L = 16

_mesh = plsc.VectorSubcoreMesh(core_axis_name="c", subcore_axis_name="s")


def _sc_add(x_hbm, pe_hbm, out_hbm, pe_v, x_v):
    wid = lax.axis_index("s") * NC + lax.axis_index("c")
    base = wid * TW

    def do_block(blk, _):
        t0 = base + blk * BTR
        pltpu.sync_copy(pe_hbm.at[pl.ds(t0, BTR), :], pe_v)
        pltpu.sync_copy(x_hbm.at[:, pl.ds(t0, BTR), :], x_v)

        def chunk(j, _):
            r = j // (D // L)
            o = (j % (D // L)) * L
            pe_vec = pe_v[r, pl.ds(o, L)]
            for b in range(B):
                x_v[b, r, pl.ds(o, L)] += pe_vec
            return 0

        lax.fori_loop(0, BTR * D // L, chunk, 0, unroll=4)
        pltpu.sync_copy(x_v, out_hbm.at[:, pl.ds(t0, BTR), :])
        return 0

    lax.fori_loop(0, NB, do_block, 0)


_sc_call = functools.partial(
    pl.kernel,
    out_type=jax.ShapeDtypeStruct((B, T, D), jnp.float32),
    mesh=_mesh,
    scratch_types=[
        pltpu.VMEM((BTR, D), jnp.float32),
        pltpu.VMEM((B, BTR, D), jnp.float32),
    ],
)(_sc_add)


def kernel(x, pe_table):
    return _sc_call(x, pe_table[:T])


# speedup vs baseline: 3.6127x; 1.5417x over previous
import functools
import jax
import jax.numpy as jnp
from jax import lax
from jax.experimental import pallas as pl
from jax.experimental.pallas import tpu as pltpu, tpu_sc as plsc

B, T, D = 4, 8192, 1024
NC, NS = 2, 16
NW = NC * NS
TW = T // NW
BTR = 8
NB = TW // BTR
L = 16

_mesh = plsc.VectorSubcoreMesh(core_axis_name="c", subcore_axis_name="s")


def _sc_add(x_hbm, pe_hbm, out_hbm, pe_v, x_v,
            sem_pe0, sem_pe1, sem_x0, sem_x1, sem_o0, sem_o1):
    sem_pe = (sem_pe0, sem_pe1)
    sem_x = (sem_x0, sem_x1)
    sem_o = (sem_o0, sem_o1)
    wid = lax.axis_index("s") * NC + lax.axis_index("c")
    base = wid * TW

    def issue_in(blk):
        slot = blk % 2
        t0 = base + blk * BTR
        dpe = pltpu.async_copy(pe_hbm.at[pl.ds(t0, BTR), :], pe_v.at[slot],
                               sem_pe[slot])
        dx = pltpu.async_copy(x_hbm.at[:, pl.ds(t0, BTR), :], x_v.at[slot],
                              sem_x[slot])
        return dpe, dx

    def compute(slot):
        pe_s = pe_v.at[slot]
        x_s = x_v.at[slot]

        def chunk(j, _):
            r = j // (D // L)
            o = (j % (D // L)) * L
            pe_vec = pe_s[r, pl.ds(o, L)]
            for b in range(B):
                x_s[b, r, pl.ds(o, L)] += pe_vec
            return 0

        lax.fori_loop(0, BTR * D // L, chunk, 0, unroll=4)

    descs_in = {}
    descs_out = {}
    descs_in[0] = issue_in(0)
    for blk in range(NB):
        slot = blk % 2
        if blk + 1 < NB:
            if blk - 1 >= 0:
                descs_out.pop(blk - 1).wait()
            descs_in[blk + 1] = issue_in(blk + 1)
        dpe, dx = descs_in.pop(blk)
        dpe.wait()
        dx.wait()
        compute(slot)
        t0 = base + blk * BTR
        descs_out[blk] = pltpu.async_copy(
            x_v.at[slot], out_hbm.at[:, pl.ds(t0, BTR), :], sem_o[slot])
    descs_out.pop(NB - 2).wait()
    descs_out.pop(NB - 1).wait()


_sc_call = functools.partial(
    pl.kernel,
    out_type=jax.ShapeDtypeStruct((B, T, D), jnp.float32),
    mesh=_mesh,
    scratch_types=[
        pltpu.VMEM((2, BTR, D), jnp.float32),
        pltpu.VMEM((2, B, BTR, D), jnp.float32),
        pltpu.SemaphoreType.DMA,
        pltpu.SemaphoreType.DMA,
        pltpu.SemaphoreType.DMA,
        pltpu.SemaphoreType.DMA,
        pltpu.SemaphoreType.DMA,
        pltpu.SemaphoreType.DMA,
    ],
)(_sc_add)


def kernel(x, pe_table):
    return _sc_call(x, pe_table[:T])


# speedup vs baseline: 4.5207x; 1.2513x over previous
import functools
import jax
import jax.numpy as jnp
from jax import lax
from jax.experimental import pallas as pl
from jax.experimental.pallas import tpu as pltpu, tpu_sc as plsc

B, T, D = 4, 8192, 1024
NC, NS = 2, 16
NW = NC * NS
TW = T // NW
BTR = 8
NB = TW // BTR
L = 16
NX = 3
NP = 2


def _sc_add(x_hbm, pe_hbm, out_hbm, pe_v, x_v, *sems):
    sem_pe = sems[0:NP]
    sem_x = sems[NP:NP + NX]
    sem_o = sems[NP + NX:NP + 2 * NX]
    wid = lax.axis_index("s") * NC + lax.axis_index("c")
    base = wid * TW

    def issue_in(blk):
        t0 = base + blk * BTR
        dpe = pltpu.async_copy(pe_hbm.at[pl.ds(t0, BTR), :],
                               pe_v.at[blk % NP], sem_pe[blk % NP])
        dx = pltpu.async_copy(x_hbm.at[:, pl.ds(t0, BTR), :],
                              x_v.at[blk % NX], sem_x[blk % NX])
        return dpe, dx

    def compute(blk):
        pe_s = pe_v.at[blk % NP]
        x_s = x_v.at[blk % NX]

        def chunk(j, _):
            r = j // (D // L)
            o = (j % (D // L)) * L
            pe_vec = pe_s[r, pl.ds(o, L)]
            for b in range(B):
                x_s[b, r, pl.ds(o, L)] += pe_vec
            return 0

        lax.fori_loop(0, BTR * D // L, chunk, 0, unroll=4)

    descs_in = {0: issue_in(0)}
    descs_out = {}
    for blk in range(NB):
        nxt = blk + 1
        if nxt < NB:
            prev_user = nxt - NX
            if prev_user >= 0:
                descs_out.pop(prev_user).wait()
            descs_in[nxt] = issue_in(nxt)
        dpe, dx = descs_in.pop(blk)
        dpe.wait()
        dx.wait()
        compute(blk)
        t0 = base + blk * BTR
        descs_out[blk] = pltpu.async_copy(
            x_v.at[blk % NX], out_hbm.at[:, pl.ds(t0, BTR), :],
            sem_o[blk % NX])
    for blk in sorted(descs_out):
        descs_out.pop(blk).wait()


_mesh = plsc.VectorSubcoreMesh(core_axis_name="c", subcore_axis_name="s")

_sc_call = functools.partial(
    pl.kernel,
    out_type=jax.ShapeDtypeStruct((B, T, D), jnp.float32),
    mesh=_mesh,
    scratch_types=(
        [pltpu.VMEM((NP, BTR, D), jnp.float32),
         pltpu.VMEM((NX, B, BTR, D), jnp.float32)]
        + [pltpu.SemaphoreType.DMA] * (NP + 2 * NX)
    ),
)(_sc_add)


def kernel(x, pe_table):
    return _sc_call(x, pe_table[:T])


# speedup vs baseline: 4.8103x; 1.0641x over previous
import functools
import jax
import jax.numpy as jnp
from jax import lax
from jax.experimental import pallas as pl
from jax.experimental.pallas import tpu as pltpu, tpu_sc as plsc

B, T, D = 4, 8192, 1024
NC, NS = 2, 16
NW = NC * NS
TW = T // NW
BTR = 8
NB = TW // BTR
L = 16
NX = 3
NP = 2


def _sc_add(x_hbm, pe_hbm, out_hbm, pe_v, x_v, *sems):
    sem_pe = sems[0:NP]
    sem_x = sems[NP:NP + NX]
    sem_o = sems[NP + NX:NP + 2 * NX]
    wid = lax.axis_index("s") * NC + lax.axis_index("c")
    base = wid * TW

    def issue_in(blk):
        t0 = base + blk * BTR
        dpe = pltpu.async_copy(pe_hbm.at[pl.ds(t0, BTR), :],
                               pe_v.at[blk % NP], sem_pe[blk % NP])
        dx = pltpu.async_copy(x_hbm.at[:, pl.ds(t0, BTR), :],
                              x_v.at[blk % NX], sem_x[blk % NX])
        return dpe, dx

    def compute(blk):
        pe_s = pe_v.at[blk % NP]
        x_s = x_v.at[blk % NX]

        def row(r, _):
            def chunk(j, _):
                o = j * L
                pe_vec = pe_s[r, pl.ds(o, L)]
                for b in range(B):
                    plsc.addupdate(x_s.at[b, r, pl.ds(o, L)], pe_vec)
                return 0

            lax.fori_loop(0, D // L, chunk, 0, unroll=8)
            return 0

        lax.fori_loop(0, BTR, row, 0)

    descs_in = {0: issue_in(0)}
    descs_out = {}
    for blk in range(NB):
        nxt = blk + 1
        if nxt < NB:
            prev_user = nxt - NX
            if prev_user >= 0:
                descs_out.pop(prev_user).wait()
            descs_in[nxt] = issue_in(nxt)
        dpe, dx = descs_in.pop(blk)
        dpe.wait()
        dx.wait()
        compute(blk)
        t0 = base + blk * BTR
        descs_out[blk] = pltpu.async_copy(
            x_v.at[blk % NX], out_hbm.at[:, pl.ds(t0, BTR), :],
            sem_o[blk % NX])
    for blk in sorted(descs_out):
        descs_out.pop(blk).wait()


_mesh = plsc.VectorSubcoreMesh(core_axis_name="c", subcore_axis_name="s")

_sc_call = functools.partial(
    pl.kernel,
    out_type=jax.ShapeDtypeStruct((B, T, D), jnp.float32),
    mesh=_mesh,
    scratch_types=(
        [pltpu.VMEM((NP, BTR, D), jnp.float32),
         pltpu.VMEM((NX, B, BTR, D), jnp.float32)]
        + [pltpu.SemaphoreType.DMA] * (NP + 2 * NX)
    ),
)(_sc_add)


def kernel(x, pe_table):
    return _sc_call(x, pe_table[:T])
